# Initial kernel scaffold; baseline (speedup 1.0000x reference)
import jax
import jax.numpy as jnp
from jax.experimental import pallas as pl


def kernel(x, edge_index, W1, b1, beta):
    raise NotImplementedError("write your pallas kernel here")


# speedup vs baseline: 4.2634x; 4.2634x over previous
import functools

import jax
import jax.numpy as jnp
from jax import lax
from jax.experimental import pallas as pl
from jax.experimental.pallas import tpu as pltpu
from jax.experimental.pallas import tpu_sc as plsc

N = 10000
E = 160000
DIN = 512
D = 256
DH = 128
NC = 2
NS = 16
L = 16

RB = 1000

_mesh = lambda: plsc.VectorSubcoreMesh(core_axis_name="c", subcore_axis_name="s")


def _feat_body(x_ref, wt_ref, b_ref, hn_ref, hn2_ref, nrm_ref):
    h = jnp.dot(x_ref[...], wt_ref[...], preferred_element_type=jnp.float32)
    h = jnp.maximum(h + b_ref[...], 0.0)
    nrm = jnp.sqrt(jnp.sum(h * h, axis=1, keepdims=True))
    hn = h * (1.0 / jnp.maximum(nrm, 1e-12))
    hn_ref[...] = hn
    hn2_ref[0] = hn[:, :DH]
    hn2_ref[1] = hn[:, DH:]
    nrm_ref[...] = nrm


def _features(x, w1t, b1):
    return pl.pallas_call(
        _feat_body,
        grid=(N // RB,),
        in_specs=[
            pl.BlockSpec((RB, DIN), lambda i: (i, 0)),
            pl.BlockSpec((DIN, D), lambda i: (0, 0)),
            pl.BlockSpec((1, D), lambda i: (0, 0)),
        ],
        out_specs=[
            pl.BlockSpec((RB, D), lambda i: (i, 0)),
            pl.BlockSpec((2, RB, DH), lambda i: (0, i, 0)),
            pl.BlockSpec((RB, 1), lambda i: (i, 0)),
        ],
        out_shape=[
            jax.ShapeDtypeStruct((N, D), jnp.float32),
            jax.ShapeDtypeStruct((2, N, DH), jnp.float32),
            jax.ShapeDtypeStruct((N, 1), jnp.float32),
        ],
    )(x, w1t, b1)


C1 = 128
NCH1 = E // C1
MAXIT1 = (NCH1 + NC * NS - 1) // (NC * NS)


def _edge_w_body(hn_hbm, src_hbm, dst_hbm, beta_hbm, w_hbm,
                 sidx, didx, srow, drow, wbuf, bbuf, sem):
    c = lax.axis_index("c")
    s = lax.axis_index("s")
    wid = s * NC + c
    pltpu.sync_copy(beta_hbm, bbuf.at[pl.ds(0, 1)])
    b = bbuf[pl.ds(0, L)][0]
    lane = lax.iota(jnp.int32, L)

    def chunk(j, _):
        cid = wid + (NC * NS) * j

        @pl.when(cid < NCH1)
        def _():
            base = cid * C1
            pltpu.sync_copy(src_hbm.at[pl.ds(base, C1)], sidx)
            pltpu.sync_copy(dst_hbm.at[pl.ds(base, C1)], didx)
            d1 = pltpu.async_copy(hn_hbm.at[sidx], srow, sem)
            d2 = pltpu.async_copy(hn_hbm.at[didx], drow, sem)
            d1.wait()
            d2.wait()

            def group(g, _):
                av = jnp.zeros((L,), jnp.float32)
                for e in range(L):
                    r = g * L + e
                    acc = jnp.zeros((L,), jnp.float32)
                    for k in range(D // L):
                        acc = acc + (srow[r, pl.ds(k * L, L)]
                                     * drow[r, pl.ds(k * L, L)])
                    av = jnp.where(lane == e, jnp.sum(acc), av)
                wbuf[pl.ds(g * L, L)] = jnp.exp(av * b)

            lax.fori_loop(0, C1 // L, group, None)
            pltpu.sync_copy(wbuf, w_hbm.at[pl.ds(base, C1)])

    lax.fori_loop(0, MAXIT1, chunk, None)


def _edge_w(hn, src, dst, beta):
    fn = functools.partial(
        pl.kernel,
        out_type=jax.ShapeDtypeStruct((E,), jnp.float32),
        mesh=_mesh(),
        compiler_params=pltpu.CompilerParams(needs_layout_passes=False),
        scratch_types=[
            pltpu.VMEM((C1,), jnp.int32),
            pltpu.VMEM((C1,), jnp.int32),
            pltpu.VMEM((C1, D), jnp.float32),
            pltpu.VMEM((C1, D), jnp.float32),
            pltpu.VMEM((C1,), jnp.float32),
            pltpu.VMEM((L,), jnp.float32),
            pltpu.SemaphoreType.DMA,
        ],
    )(_edge_w_body)
    return fn(hn, src, dst, beta)


C2 = 80
EPT = E // NS
ZR = 125


def _agg_body(hn2_hbm, src_hbm, dst_hbm, w_hbm, nrm_hbm, out_hbm,
              sidx, sidx2, didx, wv, wpv, rows, wrow, nrmv,
              zbuf1, acc_sp, den_sp, sem):
    c = lax.axis_index("c")
    s = lax.axis_index("s")
    cN = c * N
    zv = jnp.zeros((L,), jnp.float32)

    def z2(i, _):
        for k in range(DH // L):
            wrow[i, pl.ds(k * L, L)] = zv

    lax.fori_loop(0, C2, z2, None)

    def z1(i, _):
        zbuf1[pl.ds(i * L, L)] = zv

    lax.fori_loop(0, 2000 // L, z1, None)

    for j in range(7):
        pltpu.sync_copy(wrow, acc_sp.at[pl.ds(s * 625 + j * C2, C2)])
    pltpu.sync_copy(wrow.at[pl.ds(0, 65)], acc_sp.at[pl.ds(s * 625 + 560, 65)])

    @pl.when(s == 0)
    def _():
        for j in range(5):
            pltpu.sync_copy(zbuf1, den_sp.at[pl.ds(j * 2000, 2000)])

    pltpu.sync_copy(nrm_hbm, nrmv)
    plsc.subcore_barrier()

    def chunk(i, _):
        base = s * EPT + i * C2
        pltpu.sync_copy(src_hbm.at[pl.ds(base, C2)], sidx)
        pltpu.sync_copy(dst_hbm.at[pl.ds(base, C2)], didx)
        pltpu.sync_copy(w_hbm.at[pl.ds(base, C2)], wv)
        for g in range(C2 // L):
            iv = sidx[pl.ds(g * L, L)]
            sidx2[pl.ds(g * L, L)] = iv + cN
            nv = plsc.load_gather(nrmv, [iv])
            wpv[pl.ds(g * L, L)] = wv[pl.ds(g * L, L)] * nv
        pltpu.async_copy(hn2_hbm.at[sidx2], rows, sem).wait()

        def rowmul(g, _):
            wvec = wpv[pl.ds(g * L, L)]
            for e in range(L):
                r = g * L + e
                wq = wvec[e]
                for k in range(DH // L):
                    wrow[r, pl.ds(k * L, L)] = rows[r, pl.ds(k * L, L)] * wq

        lax.fori_loop(0, C2 // L, rowmul, None)
        pltpu.sync_copy(wrow, acc_sp.at[didx], add=True)
        pltpu.sync_copy(wv, den_sp.at[didx], add=True)

    lax.fori_loop(0, EPT // C2, chunk, None)
    plsc.subcore_barrier()

    def divchunk(i, _):
        @pl.when((s < NS - 1) | (i < 5))
        def _():
            row0 = s * 640 + i * C2
            pltpu.sync_copy(acc_sp.at[pl.ds(row0, C2)], wrow)
            pltpu.sync_copy(den_sp.at[pl.ds(row0, C2)], wv)

            def rdiv(g, _):
                rv = 1.0 / jnp.maximum(wv[pl.ds(g * L, L)], 1e-16)
                for e in range(L):
                    r = g * L + e
                    rq = rv[e]
                    for k in range(DH // L):
                        rows[r, pl.ds(k * L, L)] = wrow[r, pl.ds(k * L, L)] * rq

            lax.fori_loop(0, C2 // L, rdiv, None)
            pltpu.sync_copy(rows, out_hbm.at[c, pl.ds(row0, C2)])

    lax.fori_loop(0, 8, divchunk, None)


def _aggregate(hn2_flat, src, dst, w, nrm):
    fn = functools.partial(
        pl.kernel,
        out_type=jax.ShapeDtypeStruct((2, N, DH), jnp.float32),
        mesh=_mesh(),
        compiler_params=pltpu.CompilerParams(needs_layout_passes=False),
        scratch_types=[
            pltpu.VMEM((C2,), jnp.int32),
            pltpu.VMEM((C2,), jnp.int32),
            pltpu.VMEM((C2,), jnp.int32),
            pltpu.VMEM((C2,), jnp.float32),
            pltpu.VMEM((C2,), jnp.float32),
            pltpu.VMEM((C2, DH), jnp.float32),
            pltpu.VMEM((C2, DH), jnp.float32),
            pltpu.VMEM((N,), jnp.float32),
            pltpu.VMEM((2000,), jnp.float32),
            pltpu.VMEM_SHARED((N, DH), jnp.float32),
            pltpu.VMEM_SHARED((N,), jnp.float32),
            pltpu.SemaphoreType.DMA,
        ],
    )(_agg_body)
    return fn(hn2_flat, src, dst, w, nrm)


def kernel(x, edge_index, W1, b1, beta):
    hn, hn2, nrm = _features(x, W1.T, b1.reshape(1, D))
    src = edge_index[0]
    dst = edge_index[1]
    w = _edge_w(hn, src, dst, beta)
    out2 = _aggregate(jnp.reshape(hn2, (2 * N, DH)), src, dst, w,
                      jnp.reshape(nrm, (N,)))
    return jnp.concatenate([out2[0], out2[1]], axis=1)


# speedup vs baseline: 4.9371x; 1.1580x over previous
import functools

import jax
import jax.numpy as jnp
from jax import lax
from jax.experimental import pallas as pl
from jax.experimental.pallas import tpu as pltpu
from jax.experimental.pallas import tpu_sc as plsc

N = 10000
E = 160000
DIN = 512
D = 256
DH = 128
NC = 2
NS = 16
L = 16

RB = 1000

_mesh = lambda: plsc.VectorSubcoreMesh(core_axis_name="c", subcore_axis_name="s")


def _feat_body(x_ref, wt_ref, b_ref, hn_ref, hn2_ref, nrm_ref):
    h = jnp.dot(x_ref[...], wt_ref[...], preferred_element_type=jnp.float32)
    h = jnp.maximum(h + b_ref[...], 0.0)
    nrm = jnp.sqrt(jnp.sum(h * h, axis=1, keepdims=True))
    hn = h * (1.0 / jnp.maximum(nrm, 1e-12))
    hn_ref[...] = hn
    hn2_ref[0] = hn[:, :DH]
    hn2_ref[1] = hn[:, DH:]
    nrm_ref[...] = nrm


def _features(x, w1t, b1):
    return pl.pallas_call(
        _feat_body,
        grid=(N // RB,),
        in_specs=[
            pl.BlockSpec((RB, DIN), lambda i: (i, 0)),
            pl.BlockSpec((DIN, D), lambda i: (0, 0)),
            pl.BlockSpec((1, D), lambda i: (0, 0)),
        ],
        out_specs=[
            pl.BlockSpec((RB, D), lambda i: (i, 0)),
            pl.BlockSpec((2, RB, DH), lambda i: (0, i, 0)),
            pl.BlockSpec((RB, 1), lambda i: (i, 0)),
        ],
        out_shape=[
            jax.ShapeDtypeStruct((N, D), jnp.float32),
            jax.ShapeDtypeStruct((2, N, DH), jnp.float32),
            jax.ShapeDtypeStruct((N, 1), jnp.float32),
        ],
    )(x, w1t, b1)


C1 = 64
EPW = E // (NC * NS)
NCH1 = (EPW + C1 - 1) // C1
LAST1 = EPW - C1


def _edge_w_body(hn_hbm, src_hbm, dst_hbm, beta_hbm, w_hbm,
                 sidxa, didxa, srow0, drow0, srow1, drow1,
                 wbuf, bbuf, sem):
    c = lax.axis_index("c")
    s = lax.axis_index("s")
    wid = s * NC + c
    base0 = wid * EPW
    pltpu.sync_copy(beta_hbm, bbuf.at[pl.ds(0, 1)])
    pltpu.sync_copy(src_hbm.at[pl.ds(base0, EPW)], sidxa)
    pltpu.sync_copy(dst_hbm.at[pl.ds(base0, EPW)], didxa)
    b = bbuf[pl.ds(0, L)][0]
    lane = lax.iota(jnp.int32, L)

    def cbase(j):
        return jnp.minimum(j * C1, LAST1)

    def start(j, srow, drow):
        bs = cbase(j)
        pltpu.async_copy(hn_hbm.at[sidxa.at[pl.ds(bs, C1)]], srow, sem)
        pltpu.async_copy(hn_hbm.at[didxa.at[pl.ds(bs, C1)]], drow, sem)

    def drain(srow, drow):
        pltpu.make_async_copy(hn_hbm.at[pl.ds(0, C1)], srow, sem).wait()
        pltpu.make_async_copy(hn_hbm.at[pl.ds(0, C1)], drow, sem).wait()

    def compute(j, srow, drow):
        bs = cbase(j)

        def group(g, _):
            av = jnp.zeros((L,), jnp.float32)
            for e in range(L):
                r = g * L + e
                acc = jnp.zeros((L,), jnp.float32)
                for k in range(D // L):
                    acc = acc + (srow[r, pl.ds(k * L, L)]
                                 * drow[r, pl.ds(k * L, L)])
                av = jnp.where(lane == e, jnp.sum(acc), av)
            wbuf[pl.ds(g * L, L)] = jnp.exp(av * b)

        lax.fori_loop(0, C1 // L, group, None)
        pltpu.sync_copy(wbuf, w_hbm.at[pl.ds(base0 + bs, C1)])

    start(0, srow0, drow0)

    def pair(i, _):
        jb = 2 * i + 1

        @pl.when(jb < NCH1)
        def _():
            start(jb, srow1, drow1)
        drain(srow0, drow0)
        compute(2 * i, srow0, drow0)

        @pl.when(2 * i + 2 < NCH1)
        def _():
            start(2 * i + 2, srow0, drow0)

        @pl.when(jb < NCH1)
        def _():
            drain(srow1, drow1)
            compute(jb, srow1, drow1)

    lax.fori_loop(0, (NCH1 + 1) // 2, pair, None)


def _edge_w(hn, src, dst, beta):
    fn = functools.partial(
        pl.kernel,
        out_type=jax.ShapeDtypeStruct((E,), jnp.float32),
        mesh=_mesh(),
        compiler_params=pltpu.CompilerParams(needs_layout_passes=False),
        scratch_types=[
            pltpu.VMEM((EPW,), jnp.int32),
            pltpu.VMEM((EPW,), jnp.int32),
            pltpu.VMEM((C1, D), jnp.float32),
            pltpu.VMEM((C1, D), jnp.float32),
            pltpu.VMEM((C1, D), jnp.float32),
            pltpu.VMEM((C1, D), jnp.float32),
            pltpu.VMEM((C1,), jnp.float32),
            pltpu.VMEM((L,), jnp.float32),
            pltpu.SemaphoreType.DMA,
        ],
    )(_edge_w_body)
    return fn(hn, src, dst, beta)


C2 = 80
EPT = E // NS
ZR = 125


def _agg_body(hn2_hbm, src_hbm, dst_hbm, w_hbm, nrm_hbm, out_hbm,
              sidx, sidx2, didx, wv, wpv, rows, wrow, nrmv,
              zbuf1, acc_sp, den_sp, sem):
    c = lax.axis_index("c")
    s = lax.axis_index("s")
    cN = c * N
    zv = jnp.zeros((L,), jnp.float32)

    def z2(i, _):
        for k in range(DH // L):
            wrow[i, pl.ds(k * L, L)] = zv

    lax.fori_loop(0, C2, z2, None)

    def z1(i, _):
        zbuf1[pl.ds(i * L, L)] = zv

    lax.fori_loop(0, 2000 // L, z1, None)

    for j in range(7):
        pltpu.sync_copy(wrow, acc_sp.at[pl.ds(s * 625 + j * C2, C2)])
    pltpu.sync_copy(wrow.at[pl.ds(0, 65)], acc_sp.at[pl.ds(s * 625 + 560, 65)])

    @pl.when(s == 0)
    def _():
        for j in range(5):
            pltpu.sync_copy(zbuf1, den_sp.at[pl.ds(j * 2000, 2000)])

    pltpu.sync_copy(nrm_hbm, nrmv)
    plsc.subcore_barrier()

    def chunk(i, _):
        base = s * EPT + i * C2
        pltpu.sync_copy(src_hbm.at[pl.ds(base, C2)], sidx)
        pltpu.sync_copy(dst_hbm.at[pl.ds(base, C2)], didx)
        pltpu.sync_copy(w_hbm.at[pl.ds(base, C2)], wv)
        for g in range(C2 // L):
            iv = sidx[pl.ds(g * L, L)]
            sidx2[pl.ds(g * L, L)] = iv + cN
            nv = plsc.load_gather(nrmv, [iv])
            wpv[pl.ds(g * L, L)] = wv[pl.ds(g * L, L)] * nv
        pltpu.async_copy(hn2_hbm.at[sidx2], rows, sem).wait()

        def rowmul(g, _):
            wvec = wpv[pl.ds(g * L, L)]
            for e in range(L):
                r = g * L + e
                wq = wvec[e]
                for k in range(DH // L):
                    wrow[r, pl.ds(k * L, L)] = rows[r, pl.ds(k * L, L)] * wq

        lax.fori_loop(0, C2 // L, rowmul, None)
        pltpu.sync_copy(wrow, acc_sp.at[didx], add=True)
        pltpu.sync_copy(wv, den_sp.at[didx], add=True)

    lax.fori_loop(0, EPT // C2, chunk, None)
    plsc.subcore_barrier()

    def divchunk(i, _):
        @pl.when((s < NS - 1) | (i < 5))
        def _():
            row0 = s * 640 + i * C2
            pltpu.sync_copy(acc_sp.at[pl.ds(row0, C2)], wrow)
            pltpu.sync_copy(den_sp.at[pl.ds(row0, C2)], wv)

            def rdiv(g, _):
                rv = 1.0 / jnp.maximum(wv[pl.ds(g * L, L)], 1e-16)
                for e in range(L):
                    r = g * L + e
                    rq = rv[e]
                    for k in range(DH // L):
                        rows[r, pl.ds(k * L, L)] = wrow[r, pl.ds(k * L, L)] * rq

            lax.fori_loop(0, C2 // L, rdiv, None)
            pltpu.sync_copy(rows, out_hbm.at[c, pl.ds(row0, C2)])

    lax.fori_loop(0, 8, divchunk, None)


def _aggregate(hn2_flat, src, dst, w, nrm):
    fn = functools.partial(
        pl.kernel,
        out_type=jax.ShapeDtypeStruct((2, N, DH), jnp.float32),
        mesh=_mesh(),
        compiler_params=pltpu.CompilerParams(needs_layout_passes=False),
        scratch_types=[
            pltpu.VMEM((C2,), jnp.int32),
            pltpu.VMEM((C2,), jnp.int32),
            pltpu.VMEM((C2,), jnp.int32),
            pltpu.VMEM((C2,), jnp.float32),
            pltpu.VMEM((C2,), jnp.float32),
            pltpu.VMEM((C2, DH), jnp.float32),
            pltpu.VMEM((C2, DH), jnp.float32),
            pltpu.VMEM((N,), jnp.float32),
            pltpu.VMEM((2000,), jnp.float32),
            pltpu.VMEM_SHARED((N, DH), jnp.float32),
            pltpu.VMEM_SHARED((N,), jnp.float32),
            pltpu.SemaphoreType.DMA,
        ],
    )(_agg_body)
    return fn(hn2_flat, src, dst, w, nrm)


def kernel(x, edge_index, W1, b1, beta):
    hn, hn2, nrm = _features(x, W1.T, b1.reshape(1, D))
    src = edge_index[0]
    dst = edge_index[1]
    w = _edge_w(hn, src, dst, beta)
    out2 = _aggregate(jnp.reshape(hn2, (2 * N, DH)), src, dst, w,
                      jnp.reshape(nrm, (N,)))
    return jnp.concatenate([out2[0], out2[1]], axis=1)
